# baseline (speedup 1.0000x reference)
import functools

import jax
import jax.numpy as jnp
from jax import lax
from jax.experimental import pallas as pl
from jax.experimental.pallas import tpu as pltpu
from jax.experimental.pallas import tpu_sc as plsc

SEQ = 200
D = 64
BATCH = 1024

NA, NB = 128, 72
NC, NS = 2, 16
NW = NC * NS
SEQ_PER_W = BATCH // NW
IDX_PER_W = SEQ_PER_W * SEQ
TOTAL_ROWS = BATCH * SEQ


def _make_kernel():
    mesh = plsc.VectorSubcoreMesh(core_axis_name="c", subcore_axis_name="s")

    @functools.partial(
        pl.kernel,
        out_type=jax.ShapeDtypeStruct((TOTAL_ROWS, D), jnp.float32),
        mesh=mesh,
        compiler_params=pltpu.CompilerParams(use_tc_tiling_on_sc=False),
        scratch_types=[
            pltpu.VMEM((IDX_PER_W,), jnp.int32),
            pltpu.VMEM((NA, D), jnp.float32),
            pltpu.VMEM((NB, D), jnp.float32),
            pltpu.VMEM((SEQ, D), jnp.float32),
            pltpu.SemaphoreType.DMA,
            pltpu.SemaphoreType.DMA,
        ],
    )
    def k(idx_hbm, word_hbm, pos_hbm, out_hbm,
          idx_v, rows_a, rows_b, pos_v, sem_a, sem_b):
        wid = lax.axis_index("s") * NC + lax.axis_index("c")
        pltpu.sync_copy(pos_hbm, pos_v)
        pltpu.sync_copy(idx_hbm.at[pl.ds(wid * IDX_PER_W, IDX_PER_W)], idx_v)

        def gather_a(s, sem):
            return pltpu.make_async_copy(
                word_hbm.at[idx_v.at[pl.ds(s * SEQ, NA)]], rows_a, sem)

        def add_store(rows_v, n, pos_off, out_base):
            def add_row(i, _):
                for c in range(D // 16):
                    sl = pl.ds(c * 16, 16)
                    rows_v[i, sl] = rows_v[i, sl] + pos_v[pos_off + i, sl]
                return 0

            lax.fori_loop(0, n, add_row, 0, unroll=4)
            pltpu.sync_copy(rows_v, out_hbm.at[pl.ds(out_base, n)])

        gather_a(0, sem_a).start()

        def seq_body(s, _):
            base = (wid * SEQ_PER_W + s) * SEQ
            hb = pltpu.make_async_copy(
                word_hbm.at[idx_v.at[pl.ds(s * SEQ + NA, NB)]], rows_b, sem_b)
            hb.start()
            gather_a(s, sem_a).wait()
            add_store(rows_a, NA, 0, base)

            @pl.when(s + 1 < SEQ_PER_W)
            def _():
                gather_a(s + 1, sem_a).start()

            hb.wait()
            add_store(rows_b, NB, NA, base + NA)
            return 0

        lax.fori_loop(0, SEQ_PER_W, seq_body, 0)

    return k


_kernel = _make_kernel()


@jax.jit
def kernel(inputs, word_table, pos_table):
    idx = inputs.astype(jnp.int32).reshape(TOTAL_ROWS)
    out = _kernel(idx, word_table, pos_table)
    return out.reshape(BATCH, SEQ, D)
